# baseline (speedup 1.0000x reference)
import jax
import jax.numpy as jnp
from jax import lax
from jax.experimental import pallas as pl
from jax.experimental.pallas import tpu as pltpu
from jax.experimental.pallas import tpu_sc as plsc

CLS_ID = 1
B = 16
L = 4096
LP1 = L + 1
NLANE = 16
OUT_PAD = 33 * 128
NCHUNK = 4
CW = L // NCHUNK


def _body(values_hbm, lengths_hbm, out_hbm, nl_hbm,
          in_v, out_v, len_v, nl_v, sem_in, sem_out):
    row = lax.axis_index("s")
    in_handles = [
        pltpu.async_copy(
            values_hbm.at[row, pl.ds(0, CW)], in_v.at[pl.ds(0, CW)], sem_in
        )
    ]
    pltpu.sync_copy(lengths_hbm, len_v)
    lane = lax.iota(jnp.int32, NLANE)
    len_vec = len_v[...]
    my_len = jnp.sum(jnp.where(lane == row, len_vec, 0))

    for k in range(NCHUNK):
        if k + 1 < NCHUNK:
            off = (k + 1) * CW
            in_handles.append(
                pltpu.async_copy(
                    values_hbm.at[row, pl.ds(off, CW)],
                    in_v.at[pl.ds(off, CW)],
                    sem_in,
                )
            )
        in_handles[k].wait()

        @plsc.parallel_loop(k * CW, (k + 1) * CW, step=NLANE, unroll=8)
        def _shift(j):
            v = in_v[pl.ds(j, NLANE)]
            out_v[pl.ds(j + 1, NLANE)] = jnp.where(lane + j < my_len, v, 0)

        if k == 0:
            head = out_v[pl.ds(0, NLANE)]
            out_v[pl.ds(0, NLANE)] = jnp.where(lane == 0, CLS_ID, head)

    pltpu.sync_copy(out_v, out_hbm.at[row])

    @pl.when(row == 0)
    def _newlen():
        nl_v[...] = len_vec + 1
        pltpu.sync_copy(nl_v, nl_hbm)


_mesh = plsc.VectorSubcoreMesh(
    core_axis_name="c", subcore_axis_name="s", num_cores=1
)

_prepend = pl.kernel(
    _body,
    out_type=[
        jax.ShapeDtypeStruct((B, OUT_PAD), jnp.int32),
        jax.ShapeDtypeStruct((B,), jnp.int32),
    ],
    mesh=_mesh,
    compiler_params=pltpu.CompilerParams(
        needs_layout_passes=False, skip_device_barrier=True
    ),
    scratch_types=[
        pltpu.VMEM((L,), jnp.int32),
        pltpu.VMEM((OUT_PAD,), jnp.int32),
        pltpu.VMEM((NLANE,), jnp.int32),
        pltpu.VMEM((NLANE,), jnp.int32),
        pltpu.SemaphoreType.DMA,
        pltpu.SemaphoreType.DMA,
    ],
)


def kernel(values, lengths):
    out_pad, new_lengths = _prepend(
        values.astype(jnp.int32), lengths.astype(jnp.int32)
    )
    out = out_pad[:, :LP1].astype(values.dtype)
    return out, new_lengths.astype(lengths.dtype)
